# baseline (speedup 1.0000x reference)
import functools

import jax
import jax.numpy as jnp
from jax import lax
from jax.experimental import pallas as pl
from jax.experimental.pallas import tpu as pltpu
from jax.experimental.pallas import tpu_sc as plsc

D_MODEL = 1024
EPS = 1e-5
NUM_CORES = 2
NUM_SUBCORES = 16
NW = NUM_CORES * NUM_SUBCORES
CHUNK = 32
NSLICE = 4
BLOCK_ROWS = 256


def _sc_gather(tokens):
    per_w = tokens // NW
    nchunk = per_w // CHUNK
    mesh = plsc.VectorSubcoreMesh(core_axis_name="c", subcore_axis_name="s")

    @functools.partial(
        pl.kernel,
        out_type=jax.ShapeDtypeStruct((tokens, D_MODEL), jnp.float32),
        mesh=mesh,
        compiler_params=pltpu.CompilerParams(needs_layout_passes=False),
        scratch_types=[
            pltpu.VMEM((per_w,), jnp.int32),
            pltpu.VMEM((CHUNK, D_MODEL), jnp.float32),
            pltpu.VMEM((CHUNK, D_MODEL), jnp.float32),
            pltpu.SemaphoreType.DMA,
            pltpu.SemaphoreType.DMA,
            pltpu.SemaphoreType.DMA,
            pltpu.SemaphoreType.DMA,
        ],
    )
    def body(ids_hbm, table_hbm, out_hbm, idx_v, rows0, rows1,
             gs0, gs1, os0, os1):
        wid = lax.axis_index("s") * NUM_CORES + lax.axis_index("c")
        base = wid * per_w
        pltpu.sync_copy(ids_hbm.at[pl.ds(base, per_w)], idx_v)

        def start_gather(g, buf, sem):
            pltpu.async_copy(
                table_hbm.at[idx_v.at[pl.ds(g * CHUNK, CHUNK)]], buf, sem)

        def wait_gather(g, buf, sem):
            pltpu.make_async_copy(
                table_hbm.at[idx_v.at[pl.ds(g * CHUNK, CHUNK)]], buf, sem
            ).wait()

        start_gather(0, rows0, gs0)
        start_gather(1, rows1, gs1)

        def pair_body(i, _):
            for b, (buf, gs, osm) in enumerate(
                    ((rows0, gs0, os0), (rows1, gs1, os1))):
                g = i * 2 + b
                wait_gather(g, buf, gs)
                dst = out_hbm.at[pl.ds(base + g * CHUNK, CHUNK)]
                pltpu.async_copy(buf, dst, osm)
                pltpu.make_async_copy(buf, dst, osm).wait()

                @pl.when(g + 2 < nchunk)
                def _():
                    start_gather(g + 2, buf, gs)
            return 0

        lax.fori_loop(0, nchunk // 2, pair_body, 0)

    return body


def _tc_layernorm(tokens):
    def ln_body(x_ref, g_ref, b_ref, o_ref):
        x = x_ref[...]
        mean = jnp.mean(x, axis=-1, keepdims=True)
        var = jnp.mean((x - mean) * (x - mean), axis=-1, keepdims=True)
        h = (x - mean) * lax.rsqrt(var + EPS)
        o_ref[...] = h * g_ref[...] + b_ref[...]

    return pl.pallas_call(
        ln_body,
        grid=(tokens // BLOCK_ROWS,),
        in_specs=[
            pl.BlockSpec((BLOCK_ROWS, D_MODEL), lambda i: (i, 0)),
            pl.BlockSpec((1, D_MODEL), lambda i: (0, 0)),
            pl.BlockSpec((1, D_MODEL), lambda i: (0, 0)),
        ],
        out_specs=pl.BlockSpec((BLOCK_ROWS, D_MODEL), lambda i: (i, 0)),
        out_shape=jax.ShapeDtypeStruct((tokens, D_MODEL), jnp.float32),
    )


def kernel(input_ids, table, gamma, beta):
    b, s = input_ids.shape
    ids = input_ids.reshape(-1).astype(jnp.int32)
    tokens = b * s
    t = tokens // NSLICE
    gather = _sc_gather(t)
    ln = _tc_layernorm(t)
    g2 = gamma.reshape(1, D_MODEL)
    b2 = beta.reshape(1, D_MODEL)
    outs = [
        ln(gather(lax.dynamic_slice_in_dim(ids, k * t, t), table), g2, b2)
        for k in range(NSLICE)
    ]
    out = jnp.concatenate(outs, axis=0)
    return out.reshape(b, s, D_MODEL)


# speedup vs baseline: 1.3870x; 1.3870x over previous
import functools

import jax
import jax.numpy as jnp
from jax import lax
from jax.experimental import pallas as pl
from jax.experimental.pallas import tpu as pltpu
from jax.experimental.pallas import tpu_sc as plsc

D_MODEL = 1024
EPS = 1e-5
NUM_CORES = 2
NUM_SUBCORES = 16
NW = NUM_CORES * NUM_SUBCORES
CHUNK = 32
NSLICE = 4
BLOCK_ROWS = 256


def _sc_gather(tokens):
    per_w = tokens // NW
    nchunk = per_w // CHUNK
    mesh = plsc.VectorSubcoreMesh(core_axis_name="c", subcore_axis_name="s")

    @functools.partial(
        pl.kernel,
        out_type=jax.ShapeDtypeStruct((tokens, D_MODEL), jnp.float32),
        mesh=mesh,
        compiler_params=pltpu.CompilerParams(needs_layout_passes=False),
        scratch_types=[
            pltpu.VMEM((per_w,), jnp.int32),
            pltpu.VMEM((CHUNK, D_MODEL), jnp.float32),
            pltpu.VMEM((CHUNK, D_MODEL), jnp.float32),
            pltpu.SemaphoreType.DMA,
            pltpu.SemaphoreType.DMA,
            pltpu.SemaphoreType.DMA,
            pltpu.SemaphoreType.DMA,
        ],
    )
    def body(ids_hbm, table_hbm, out_hbm, idx_v, rows0, rows1,
             gs0, gs1, os0, os1):
        wid = lax.axis_index("s") * NUM_CORES + lax.axis_index("c")
        base = wid * per_w
        pltpu.sync_copy(ids_hbm.at[pl.ds(base, per_w)], idx_v)

        def start_gather(g, buf, sem):
            pltpu.async_copy(
                table_hbm.at[idx_v.at[pl.ds(g * CHUNK, CHUNK)]], buf, sem)

        def wait_gather(g, buf, sem):
            pltpu.make_async_copy(
                table_hbm.at[idx_v.at[pl.ds(g * CHUNK, CHUNK)]], buf, sem
            ).wait()

        start_gather(0, rows0, gs0)
        start_gather(1, rows1, gs1)

        def pair_body(i, _):
            for b, (buf, gs, osm) in enumerate(
                    ((rows0, gs0, os0), (rows1, gs1, os1))):
                g = i * 2 + b
                wait_gather(g, buf, gs)
                dst = out_hbm.at[pl.ds(base + g * CHUNK, CHUNK)]
                pltpu.async_copy(buf, dst, osm)
                pltpu.make_async_copy(buf, dst, osm).wait()

                @pl.when(g + 2 < nchunk)
                def _():
                    start_gather(g + 2, buf, gs)
            return 0

        lax.fori_loop(0, nchunk // 2, pair_body, 0)

    return body


def _tc_layernorm(tokens):
    def ln_body(x_ref, g_ref, b_ref, o_ref):
        x = x_ref[...]
        mean = jnp.mean(x, axis=-1, keepdims=True)
        var = jnp.mean((x - mean) * (x - mean), axis=-1, keepdims=True)
        h = (x - mean) * lax.rsqrt(var + EPS)
        o_ref[...] = h * g_ref[...] + b_ref[...]

    return pl.pallas_call(
        ln_body,
        grid=(tokens // BLOCK_ROWS,),
        in_specs=[
            pl.BlockSpec((BLOCK_ROWS, D_MODEL), lambda i: (i, 0)),
            pl.BlockSpec((1, D_MODEL), lambda i: (0, 0)),
            pl.BlockSpec((1, D_MODEL), lambda i: (0, 0)),
        ],
        out_specs=pl.BlockSpec((BLOCK_ROWS, D_MODEL), lambda i: (i, 0)),
        out_shape=jax.ShapeDtypeStruct((tokens, D_MODEL), jnp.float32),
    )


def kernel(input_ids, table, gamma, beta):
    b, s = input_ids.shape
    ids = input_ids.reshape(-1).astype(jnp.int32)
    tokens = b * s
    gathered = _sc_gather(tokens)(ids, table)
    out = _tc_layernorm(tokens)(
        gathered, gamma.reshape(1, D_MODEL), beta.reshape(1, D_MODEL))
    return out.reshape(b, s, D_MODEL)


# speedup vs baseline: 1.6305x; 1.1756x over previous
import functools

import jax
import jax.numpy as jnp
from jax import lax
from jax.experimental import pallas as pl
from jax.experimental.pallas import tpu as pltpu
from jax.experimental.pallas import tpu_sc as plsc

D_MODEL = 1024
EPS = 1e-5
NUM_CORES = 2
NUM_SUBCORES = 16
NW = NUM_CORES * NUM_SUBCORES
CHUNK = 32
NSLICE = 4
BLOCK_ROWS = 512


def _sc_gather(tokens):
    per_w = tokens // NW
    nchunk = per_w // CHUNK
    mesh = plsc.VectorSubcoreMesh(core_axis_name="c", subcore_axis_name="s")

    @functools.partial(
        pl.kernel,
        out_type=jax.ShapeDtypeStruct((tokens, D_MODEL), jnp.float32),
        mesh=mesh,
        compiler_params=pltpu.CompilerParams(needs_layout_passes=False),
        scratch_types=[
            pltpu.VMEM((per_w,), jnp.int32),
            pltpu.VMEM((CHUNK, D_MODEL), jnp.float32),
            pltpu.VMEM((CHUNK, D_MODEL), jnp.float32),
            pltpu.SemaphoreType.DMA,
            pltpu.SemaphoreType.DMA,
            pltpu.SemaphoreType.DMA,
            pltpu.SemaphoreType.DMA,
        ],
    )
    def body(ids_hbm, table_hbm, out_hbm, idx_v, rows0, rows1,
             gs0, gs1, os0, os1):
        wid = lax.axis_index("s") * NUM_CORES + lax.axis_index("c")
        base = wid * per_w
        pltpu.sync_copy(ids_hbm.at[pl.ds(base, per_w)], idx_v)

        def start_gather(g, buf, sem):
            pltpu.async_copy(
                table_hbm.at[idx_v.at[pl.ds(g * CHUNK, CHUNK)]], buf, sem)

        def wait_gather(g, buf, sem):
            pltpu.make_async_copy(
                table_hbm.at[idx_v.at[pl.ds(g * CHUNK, CHUNK)]], buf, sem
            ).wait()

        start_gather(0, rows0, gs0)
        start_gather(1, rows1, gs1)

        def pair_body(i, _):
            for b, (buf, gs, osm) in enumerate(
                    ((rows0, gs0, os0), (rows1, gs1, os1))):
                g = i * 2 + b
                wait_gather(g, buf, gs)
                dst = out_hbm.at[pl.ds(base + g * CHUNK, CHUNK)]
                pltpu.async_copy(buf, dst, osm)
                pltpu.make_async_copy(buf, dst, osm).wait()

                @pl.when(g + 2 < nchunk)
                def _():
                    start_gather(g + 2, buf, gs)
            return 0

        lax.fori_loop(0, nchunk // 2, pair_body, 0)

    return body


def _tc_layernorm(tokens):
    def ln_body(x_ref, g_ref, b_ref, o_ref):
        x = x_ref[...]
        mean = jnp.mean(x, axis=-1, keepdims=True)
        ex2 = jnp.mean(x * x, axis=-1, keepdims=True)
        r = lax.rsqrt(ex2 - mean * mean + EPS)
        o_ref[...] = (x - mean) * r * g_ref[...] + b_ref[...]

    return pl.pallas_call(
        ln_body,
        grid=(tokens // BLOCK_ROWS,),
        in_specs=[
            pl.BlockSpec((BLOCK_ROWS, D_MODEL), lambda i: (i, 0)),
            pl.BlockSpec((1, D_MODEL), lambda i: (0, 0)),
            pl.BlockSpec((1, D_MODEL), lambda i: (0, 0)),
        ],
        out_specs=pl.BlockSpec((BLOCK_ROWS, D_MODEL), lambda i: (i, 0)),
        out_shape=jax.ShapeDtypeStruct((tokens, D_MODEL), jnp.float32),
    )


def kernel(input_ids, table, gamma, beta):
    b, s = input_ids.shape
    ids = input_ids.reshape(-1).astype(jnp.int32)
    tokens = b * s
    gathered = _sc_gather(tokens)(ids, table)
    out = _tc_layernorm(tokens)(
        gathered, gamma.reshape(1, D_MODEL), beta.reshape(1, D_MODEL))
    return out.reshape(b, s, D_MODEL)


# speedup vs baseline: 1.7500x; 1.0733x over previous
import functools

import jax
import jax.numpy as jnp
from jax import lax
from jax.experimental import pallas as pl
from jax.experimental.pallas import tpu as pltpu
from jax.experimental.pallas import tpu_sc as plsc

D_MODEL = 1024
EPS = 1e-5
NUM_CORES = 2
NUM_SUBCORES = 16
NW = NUM_CORES * NUM_SUBCORES
CHUNK = 32
NSLICE = 4
BLOCK_ROWS = 1024


def _sc_gather(tokens):
    per_w = tokens // NW
    nchunk = per_w // CHUNK
    mesh = plsc.VectorSubcoreMesh(core_axis_name="c", subcore_axis_name="s")

    @functools.partial(
        pl.kernel,
        out_type=jax.ShapeDtypeStruct((tokens, D_MODEL), jnp.float32),
        mesh=mesh,
        compiler_params=pltpu.CompilerParams(needs_layout_passes=False),
        scratch_types=[
            pltpu.VMEM((per_w,), jnp.int32),
            pltpu.VMEM((CHUNK, D_MODEL), jnp.float32),
            pltpu.VMEM((CHUNK, D_MODEL), jnp.float32),
            pltpu.SemaphoreType.DMA,
            pltpu.SemaphoreType.DMA,
            pltpu.SemaphoreType.DMA,
            pltpu.SemaphoreType.DMA,
        ],
    )
    def body(ids_hbm, table_hbm, out_hbm, idx_v, rows0, rows1,
             gs0, gs1, os0, os1):
        wid = lax.axis_index("s") * NUM_CORES + lax.axis_index("c")
        base = wid * per_w
        pltpu.sync_copy(ids_hbm.at[pl.ds(base, per_w)], idx_v)

        def start_gather(g, buf, sem):
            pltpu.async_copy(
                table_hbm.at[idx_v.at[pl.ds(g * CHUNK, CHUNK)]], buf, sem)

        def wait_gather(g, buf, sem):
            pltpu.make_async_copy(
                table_hbm.at[idx_v.at[pl.ds(g * CHUNK, CHUNK)]], buf, sem
            ).wait()

        start_gather(0, rows0, gs0)
        start_gather(1, rows1, gs1)

        def pair_body(i, _):
            for b, (buf, gs, osm) in enumerate(
                    ((rows0, gs0, os0), (rows1, gs1, os1))):
                g = i * 2 + b
                wait_gather(g, buf, gs)
                dst = out_hbm.at[pl.ds(base + g * CHUNK, CHUNK)]
                pltpu.async_copy(buf, dst, osm)
                pltpu.make_async_copy(buf, dst, osm).wait()

                @pl.when(g + 2 < nchunk)
                def _():
                    start_gather(g + 2, buf, gs)
            return 0

        lax.fori_loop(0, nchunk // 2, pair_body, 0)

    return body


def _tc_layernorm(tokens):
    def ln_body(x_ref, g_ref, b_ref, o_ref):
        x = x_ref[...]
        mean = jnp.mean(x, axis=-1, keepdims=True)
        ex2 = jnp.mean(x * x, axis=-1, keepdims=True)
        r = lax.rsqrt(ex2 - mean * mean + EPS)
        o_ref[...] = (x - mean) * r * g_ref[...] + b_ref[...]

    return pl.pallas_call(
        ln_body,
        grid=(tokens // BLOCK_ROWS,),
        in_specs=[
            pl.BlockSpec((BLOCK_ROWS, D_MODEL), lambda i: (i, 0)),
            pl.BlockSpec((1, D_MODEL), lambda i: (0, 0)),
            pl.BlockSpec((1, D_MODEL), lambda i: (0, 0)),
        ],
        out_specs=pl.BlockSpec((BLOCK_ROWS, D_MODEL), lambda i: (i, 0)),
        out_shape=jax.ShapeDtypeStruct((tokens, D_MODEL), jnp.float32),
    )


def kernel(input_ids, table, gamma, beta):
    b, s = input_ids.shape
    ids = input_ids.reshape(-1).astype(jnp.int32)
    tokens = b * s
    gathered = _sc_gather(tokens)(ids, table)
    out = _tc_layernorm(tokens)(
        gathered, gamma.reshape(1, D_MODEL), beta.reshape(1, D_MODEL))
    return out.reshape(b, s, D_MODEL)
